# baseline (speedup 1.0000x reference)
import jax
import jax.numpy as jnp
from jax.experimental import pallas as pl
from jax.experimental.pallas import tpu as pltpu


def _body(slots_ref, masks_ref, w1_ref, b1_ref, w2_ref, b2_ref, ps_ref, pm_ref):
    m = masks_ref[...]
    n = m.shape[1]
    util = jnp.sum(m, axis=1) * (1.0 / n)
    h = util[:, :, None] * w1_ref[...][None, :, :] + b1_ref[...][None, :, :]
    h = jnp.maximum(h, 0.0)
    logit = jnp.sum(h * w2_ref[...][None, :, :], axis=-1) + b2_ref[0, 0]
    g = jax.nn.sigmoid(logit)
    ps_ref[...] = slots_ref[...] * g[:, :, None]
    pm = m * g[:, None, :]
    s = jnp.sum(pm, axis=-1, keepdims=True)
    pm_ref[...] = pm / (s + 1e-8)


def kernel(slots, masks, w1, b1, w2, b2):
    B, K, D = slots.shape
    N = masks.shape[1]
    BT = 4
    w1r = w1.reshape(1, 16)
    b1r = b1.reshape(1, 16)
    w2r = w2.reshape(1, 16)
    b2r = b2.reshape(1, 1)
    grid = (B // BT,)
    ps, pm = pl.pallas_call(
        _body,
        grid=grid,
        in_specs=[
            pl.BlockSpec((BT, K, D), lambda i: (i, 0, 0)),
            pl.BlockSpec((BT, N, K), lambda i: (i, 0, 0)),
            pl.BlockSpec((1, 16), lambda i: (0, 0)),
            pl.BlockSpec((1, 16), lambda i: (0, 0)),
            pl.BlockSpec((1, 16), lambda i: (0, 0)),
            pl.BlockSpec((1, 1), lambda i: (0, 0)),
        ],
        out_specs=[
            pl.BlockSpec((BT, K, D), lambda i: (i, 0, 0)),
            pl.BlockSpec((BT, N, K), lambda i: (i, 0, 0)),
        ],
        out_shape=[
            jax.ShapeDtypeStruct((B, K, D), jnp.float32),
            jax.ShapeDtypeStruct((B, N, K), jnp.float32),
        ],
        compiler_params=pltpu.CompilerParams(
            dimension_semantics=("arbitrary",),
        ),
    )(slots, masks, w1r, b1r, w2r, b2r)
    return (ps, pm)


# speedup vs baseline: 2.5937x; 2.5937x over previous
import jax
import jax.numpy as jnp
from jax.experimental import pallas as pl
from jax.experimental.pallas import tpu as pltpu

_K = 12
_LW = 384
_SEG = _LW // _K


def _onehot(pred):
    return jnp.where(pred, 1.0, 0.0).astype(jnp.float32)


def _body(slots_ref, masks_ref, w1_ref, b1_ref, w2_ref, b2_ref, n_inv_ref,
          ps_ref, pm_ref):
    m3 = masks_ref[...]
    bt, r, _ = m3.shape

    colsum = jnp.sum(m3, axis=1)
    jj = jax.lax.broadcasted_iota(jnp.int32, (_LW, _K), 0)
    kk = jax.lax.broadcasted_iota(jnp.int32, (_LW, _K), 1)
    e_util = _onehot(jj % _K == kk)
    util = jax.lax.dot_general(
        colsum, e_util, (((1,), (0,)), ((), ())),
        preferred_element_type=jnp.float32) * n_inv_ref[0, 0]

    h = util[:, :, None] * w1_ref[...][None, :, :] + b1_ref[...][None, :, :]
    h = jnp.maximum(h, 0.0)
    logit = jnp.sum(h * w2_ref[...][None, :, :], axis=-1) + b2_ref[0, 0]
    g = jax.nn.sigmoid(logit)

    e_g = _onehot(jj % _K == kk).T
    gpat = jax.lax.dot_general(
        g, e_g, (((1,), (0,)), ((), ())),
        preferred_element_type=jnp.float32)
    pm3 = m3 * gpat[:, None, :]

    j2 = jax.lax.broadcasted_iota(jnp.int32, (_LW, _SEG), 0)
    s2 = jax.lax.broadcasted_iota(jnp.int32, (_LW, _SEG), 1)
    e_seg = _onehot(j2 // _K == s2)
    msum = jax.lax.dot_general(
        pm3, e_seg, (((2,), (0,)), ((), ())),
        preferred_element_type=jnp.float32)
    recip = 1.0 / (msum + 1e-8)
    rexp = jax.lax.dot_general(
        recip, e_seg.T, (((2,), (0,)), ((), ())),
        preferred_element_type=jnp.float32)
    pm_ref[...] = pm3 * rexp

    ps_ref[...] = slots_ref[...] * g[:, :, None]


def kernel(slots, masks, w1, b1, w2, b2):
    B, K, D = slots.shape
    N = masks.shape[1]
    R = N * K // _LW
    BT = 8
    masks_r = masks.reshape(B, R, _LW)
    w1r = w1.reshape(1, 16)
    b1r = b1.reshape(1, 16)
    w2r = w2.reshape(1, 16)
    b2r = b2.reshape(1, 1)
    n_inv = jnp.full((1, 1), 1.0 / N, dtype=jnp.float32)
    grid = (B // BT,)
    ps, pm = pl.pallas_call(
        _body,
        grid=grid,
        in_specs=[
            pl.BlockSpec((BT, K, D), lambda i: (i, 0, 0)),
            pl.BlockSpec((BT, R, _LW), lambda i: (i, 0, 0)),
            pl.BlockSpec((1, 16), lambda i: (0, 0)),
            pl.BlockSpec((1, 16), lambda i: (0, 0)),
            pl.BlockSpec((1, 16), lambda i: (0, 0)),
            pl.BlockSpec((1, 1), lambda i: (0, 0)),
            pl.BlockSpec((1, 1), lambda i: (0, 0)),
        ],
        out_specs=[
            pl.BlockSpec((BT, K, D), lambda i: (i, 0, 0)),
            pl.BlockSpec((BT, R, _LW), lambda i: (i, 0, 0)),
        ],
        out_shape=[
            jax.ShapeDtypeStruct((B, K, D), jnp.float32),
            jax.ShapeDtypeStruct((B, R, _LW), jnp.float32),
        ],
        compiler_params=pltpu.CompilerParams(
            dimension_semantics=("arbitrary",),
        ),
    )(slots, masks_r, w1r, b1r, w2r, b2r, n_inv)
    return (ps, pm.reshape(B, N, K))
